# baseline (speedup 1.0000x reference)
import functools

import jax
import jax.numpy as jnp
from jax.experimental import pallas as pl

H = 256
F32 = jnp.float32


def _dot(a, b):
    return jnp.dot(a, b, preferred_element_type=F32)


def _block(n, target):
    b = min(n, target)
    while n % b or b % 8:
        b -= 1
    return b


def _full(shape):
    return pl.BlockSpec(shape, lambda *_: tuple(0 for _ in shape))


def _rows(bn, d):
    return pl.BlockSpec((bn, d), lambda i, *_: (i, 0))


def _weight_fuse_body(w2, b2, wu, bu, wf, bf):
    wf[...] = _dot(w2[...], wu[...])
    bf[...] = _dot(b2[...], wu[...]) + bu[...]


def _weight_fuse(w2, b2, wu, bu):
    return pl.pallas_call(
        _weight_fuse_body,
        out_shape=(jax.ShapeDtypeStruct(wu.shape, F32),
                   jax.ShapeDtypeStruct(bu.shape, F32)),
    )(w2, b2, wu, bu)


def _edge_proj_body(ea, w1, b1, wf, bf, out):
    h = jnp.maximum(_dot(ea[...], w1[...]) + b1[...], 0.0)
    out[...] = _dot(h, wf[...]) + bf[...]


def _edge_proj(ea, w1, b1, wf, bf):
    e, din = ea.shape
    dout = wf.shape[1]
    be = _block(e, 1280)
    return pl.pallas_call(
        _edge_proj_body,
        grid=(e // be,),
        in_specs=[_rows(be, din), _full(w1.shape), _full(b1.shape),
                  _full(wf.shape), _full(bf.shape)],
        out_specs=_rows(be, dout),
        out_shape=jax.ShapeDtypeStruct((e, dout), F32),
    )(ea, w1, b1, wf, bf)


def _node_enc_body(x, dummy, w1, b1, w2, b2, out):
    xv = x[...]
    xc = jnp.where(xv[:, 0:1] == -999.0, dummy[...], xv)
    h = jnp.maximum(_dot(xc, w1[...]) + b1[...], 0.0)
    out[...] = _dot(h, w2[...]) + b2[...]


def _node_enc(x, dummy, w1, b1, w2, b2):
    n, din = x.shape
    bn = _block(n, 2000)
    return pl.pallas_call(
        _node_enc_body,
        grid=(n // bn,),
        in_specs=[_rows(bn, din), _full(dummy.shape), _full(w1.shape),
                  _full(b1.shape), _full(w2.shape), _full(b2.shape)],
        out_specs=_rows(bn, H),
        out_shape=jax.ShapeDtypeStruct((n, H), F32),
    )(x, dummy, w1, b1, w2, b2)


def _lin_body(x, w, out):
    out[...] = _dot(x[...], w[...])


def _lin(x, w):
    n, din = x.shape
    dout = w.shape[1]
    bn = _block(n, 2000)
    return pl.pallas_call(
        _lin_body,
        grid=(n // bn,),
        in_specs=[_rows(bn, din), _full(w.shape)],
        out_specs=_rows(bn, dout),
        out_shape=jax.ShapeDtypeStruct((n, dout), F32),
    )(x, w)


def _edge_h_body(gd, gs, ep, w, b, out):
    pre = jnp.maximum(gd[...] + gs[...] + ep[...], 0.0)
    out[...] = _dot(pre, w[...]) + b[...]


def _edge_h(gd, gs, ep, kcol, w, b):
    e = gd.shape[0]
    be = _block(e, 1280)
    dout = w.shape[1]
    return pl.pallas_call(
        _edge_h_body,
        grid=(e // be,),
        in_specs=[_rows(be, H), _rows(be, H),
                  pl.BlockSpec((be, H), lambda i, k=kcol: (i, k)),
                  _full(w.shape), _full(b.shape)],
        out_specs=_rows(be, dout),
        out_shape=jax.ShapeDtypeStruct((e, dout), F32),
    )(gd, gs, ep, w, b)


def _edge_head_body(gs, gd, ep, w, b, lo, po):
    pre = jnp.maximum(gs[...] + gd[...] + ep[...], 0.0)
    logit = _dot(pre, w[...]) + b[...]
    lo[...] = logit
    po[...] = jax.nn.sigmoid(logit)


def _edge_head(gs, gd, ep, kcol, w, b):
    e = gs.shape[0]
    be = _block(e, 1280)
    dout = w.shape[1]
    return pl.pallas_call(
        _edge_head_body,
        grid=(e // be,),
        in_specs=[_rows(be, H), _rows(be, H),
                  pl.BlockSpec((be, H), lambda i, k=kcol: (i, k)),
                  _full(w.shape), _full(b.shape)],
        out_specs=(_rows(be, dout), _rows(be, dout)),
        out_shape=(jax.ShapeDtypeStruct((e, dout), F32),
                   jax.ShapeDtypeStruct((e, dout), F32)),
    )(gs, gd, ep, w, b)


def _edge_z_body(alpha, gq, ep, out):
    out[...] = alpha[...] * jnp.maximum(gq[...] + ep[...], 0.0)


def _edge_z(alpha, gq, ep, kcol):
    e = gq.shape[0]
    be = _block(e, 1280)
    return pl.pallas_call(
        _edge_z_body,
        grid=(e // be,),
        in_specs=[_rows(be, 1), _rows(be, H),
                  pl.BlockSpec((be, H), lambda i, k=kcol: (i, k))],
        out_specs=_rows(be, H),
        out_shape=jax.ShapeDtypeStruct((e, H), F32),
    )(alpha, gq, ep)


def _conv_post_body(s, sa, xres, w2, b2, g, b, out):
    agg = _dot(s[...], w2[...]) + sa[...] * b2[...]
    y = agg + xres[...]
    m = jnp.mean(y, axis=-1, keepdims=True)
    v = jnp.mean((y - m) * (y - m), axis=-1, keepdims=True)
    out[...] = jnp.maximum((y - m) / jnp.sqrt(v + 1e-5) * g[...] + b[...], 0.0)


def _conv_post(s, sa, xres, w2, b2, g, b):
    n = s.shape[0]
    bn = _block(n, 2000)
    return pl.pallas_call(
        _conv_post_body,
        grid=(n // bn,),
        in_specs=[_rows(bn, H), _rows(bn, 1), _rows(bn, H), _full(w2.shape),
                  _full(b2.shape), _full(g.shape), _full(b.shape)],
        out_specs=_rows(bn, H),
        out_shape=jax.ShapeDtypeStruct((n, H), F32),
    )(s, sa, xres, w2, b2, g, b)


def _node_head_body(x, w1, b1, w2, b2, lo, po):
    h = jnp.maximum(_dot(x[...], w1[...]) + b1[...], 0.0)
    logit = _dot(h, w2[...]) + b2[...]
    lo[...] = logit
    po[...] = jax.nn.softmax(logit, axis=-1)


def _node_head(x, w1, b1, w2, b2):
    n = x.shape[0]
    bn = _block(n, 2000)
    dout = w2.shape[1]
    return pl.pallas_call(
        _node_head_body,
        grid=(n // bn,),
        in_specs=[_rows(bn, H), _full(w1.shape), _full(b1.shape),
                  _full(w2.shape), _full(b2.shape)],
        out_specs=(_rows(bn, dout), _rows(bn, dout)),
        out_shape=(jax.ShapeDtypeStruct((n, dout), F32),
                   jax.ShapeDtypeStruct((n, dout), F32)),
    )(x, w1, b1, w2, b2)


def _conv_stage(xn, EP, k_att, k_msg, src, dst, p, pre, ln_g, ln_b):
    n = xn.shape[0]
    wcat = jnp.concatenate([p[pre + '_aW1'][:H], p[pre + '_aW1'][H:2 * H],
                            p[pre + '_mW1'][:H]], axis=1)
    proj = _lin(xn, wcat)
    gd = proj[:, :H][dst]
    gs = proj[:, H:][src]
    aw2 = jnp.pad(p[pre + '_aW2'], ((0, 0), (0, 7)))
    ab2 = jnp.pad(p[pre + '_ab2'], (0, 7))[None]
    h = _edge_h(gd, gs[:, :H], EP, k_att, aw2, ab2)[:, 0]
    smax = jax.ops.segment_max(h, dst, num_segments=n)
    smax = jnp.where(jnp.isfinite(smax), smax, 0.0)
    ex = jnp.exp(h - smax[dst])
    den = jax.ops.segment_sum(ex, dst, num_segments=n)
    alpha = ex / (den[dst] + 1e-16)
    z = _edge_z(alpha[:, None], gs[:, H:], EP, k_msg)
    s = jax.ops.segment_sum(z, dst, num_segments=n)
    salpha = (den / (den + 1e-16))[:, None]
    return _conv_post(s, salpha, xn, p[pre + '_mW2'], p[pre + '_mb2'][None],
                      ln_g[None], ln_b[None])


def kernel(x_in, edge_index, edge_attr, params):
    p = params
    src = edge_index[0]
    dst = edge_index[1]

    wu = jnp.concatenate([p['c1_aW1'][2 * H:], p['c1_mW1'][H:],
                          p['c2_aW1'][2 * H:], p['c2_mW1'][H:],
                          p['eh_W1'][2 * H:]], axis=1)
    bu = jnp.concatenate([p['c1_ab1'], p['c1_mb1'], p['c2_ab1'],
                          p['c2_mb1'], p['eh_b1']])[None]
    wf, bf = _weight_fuse(p['ee_W2'], p['ee_b2'][None], wu, bu)
    EP = _edge_proj(edge_attr, p['ee_W1'], p['ee_b1'][None], wf, bf)

    x = _node_enc(x_in, p['dummy'][None], p['ne_W1'], p['ne_b1'][None],
                  p['ne_W2'], p['ne_b2'][None])

    x1 = _conv_stage(x, EP, 0, 1, src, dst, p, 'c1', p['ln1_g'], p['ln1_b'])
    x2 = _conv_stage(x1, EP, 2, 3, src, dst, p, 'c2', p['ln2_g'], p['ln2_b'])

    nw2 = jnp.pad(p['nh_W2'], ((0, 0), (0, 1)))
    nb2 = jnp.concatenate([p['nh_b2'], jnp.full((1,), -1e30, F32)])[None]
    node_logits8, node_probs8 = _node_head(x2, p['nh_W1'], p['nh_b1'][None],
                                           nw2, nb2)
    node_logits = node_logits8[:, :7]
    node_probs = node_probs8[:, :7]

    wcat_h = jnp.concatenate([p['eh_W1'][:H], p['eh_W1'][H:2 * H]], axis=1)
    proj_h = _lin(x2, wcat_h)
    ga = proj_h[:, :H][src]
    gb = proj_h[:, H:][dst]
    ew2 = jnp.pad(p['eh_W2'], ((0, 0), (0, 7)))
    eb2 = jnp.pad(p['eh_b2'], (0, 7))[None]
    el8, ep8 = _edge_head(ga, gb, EP, 4, ew2, eb2)
    edge_logits = el8[:, :1]
    edge_probs = ep8[:, :1]

    return (node_logits, edge_logits, node_probs, edge_probs)


# speedup vs baseline: 1.8877x; 1.8877x over previous
import functools

import jax
import jax.numpy as jnp
from jax import lax
from jax.experimental import pallas as pl
from jax.experimental.pallas import tpu as pltpu
from jax.experimental.pallas import tpu_sc as plsc

H = 256
F32 = jnp.float32

_NC = 2
_NS = 16
_NW = _NC * _NS


def _dot(a, b):
    return jnp.dot(a, b, preferred_element_type=F32)


def _block(n, target):
    b = min(n, target)
    while n % b or b % 8:
        b -= 1
    return b


def _full(shape):
    return pl.BlockSpec(shape, lambda *_: tuple(0 for _ in shape))


def _rows(bn, d):
    return pl.BlockSpec((bn, d), lambda i, *_: (i, 0))


def _weight_fuse_body(w2, b2, wu, bu, wf, bf):
    wf[...] = _dot(w2[...], wu[...])
    bf[...] = _dot(b2[...], wu[...]) + bu[...]


def _weight_fuse(w2, b2, wu, bu):
    return pl.pallas_call(
        _weight_fuse_body,
        out_shape=(jax.ShapeDtypeStruct(wu.shape, F32),
                   jax.ShapeDtypeStruct(bu.shape, F32)),
    )(w2, b2, wu, bu)


def _edge_proj_body(ea, w1, b1, wf, bf, out):
    h = jnp.maximum(_dot(ea[...], w1[...]) + b1[...], 0.0)
    out[...] = _dot(h, wf[...]) + bf[...]


def _edge_proj(ea, w1, b1, wf, bf):
    e, din = ea.shape
    dout = wf.shape[1]
    be = _block(e, 1280)
    return pl.pallas_call(
        _edge_proj_body,
        grid=(e // be,),
        in_specs=[_rows(be, din), _full(w1.shape), _full(b1.shape),
                  _full(wf.shape), _full(bf.shape)],
        out_specs=_rows(be, dout),
        out_shape=jax.ShapeDtypeStruct((e, dout), F32),
    )(ea, w1, b1, wf, bf)


def _node_enc_body(x, dummy, w1, b1, w2, b2, out):
    xv = x[...]
    xc = jnp.where(xv[:, 0:1] == -999.0, dummy[...], xv)
    h = jnp.maximum(_dot(xc, w1[...]) + b1[...], 0.0)
    out[...] = _dot(h, w2[...]) + b2[...]


def _node_enc(x, dummy, w1, b1, w2, b2):
    n, din = x.shape
    bn = _block(n, 2000)
    return pl.pallas_call(
        _node_enc_body,
        grid=(n // bn,),
        in_specs=[_rows(bn, din), _full(dummy.shape), _full(w1.shape),
                  _full(b1.shape), _full(w2.shape), _full(b2.shape)],
        out_specs=_rows(bn, H),
        out_shape=jax.ShapeDtypeStruct((n, H), F32),
    )(x, dummy, w1, b1, w2, b2)


def _lin_body(x, w, out):
    out[...] = _dot(x[...], w[...])


def _lin(x, w):
    n, din = x.shape
    dout = w.shape[1]
    bn = _block(n, 2000)
    return pl.pallas_call(
        _lin_body,
        grid=(n // bn,),
        in_specs=[_rows(bn, din), _full(w.shape)],
        out_specs=_rows(bn, dout),
        out_shape=jax.ShapeDtypeStruct((n, dout), F32),
    )(x, w)


def _edge_h_body(gd, gs, ep, w, b, out):
    pre = jnp.maximum(gd[...] + gs[...] + ep[...], 0.0)
    out[...] = _dot(pre, w[...]) + b[...]


def _edge_h(gd, gs, ep, kcol, w, b, gcol=0):
    e = gd.shape[0]
    be = _block(e, 1280)
    dout = w.shape[1]
    return pl.pallas_call(
        _edge_h_body,
        grid=(e // be,),
        in_specs=[_rows(be, H), pl.BlockSpec((be, H), lambda i, g=gcol: (i, g)),
                  pl.BlockSpec((be, H), lambda i, k=kcol: (i, k)),
                  _full(w.shape), _full(b.shape)],
        out_specs=_rows(be, dout),
        out_shape=jax.ShapeDtypeStruct((e, dout), F32),
    )(gd, gs, ep, w, b)


def _edge_head_body(gs, gd, ep, w, b, lo, po):
    pre = jnp.maximum(gs[...] + gd[...] + ep[...], 0.0)
    logit = _dot(pre, w[...]) + b[...]
    lo[...] = logit
    po[...] = jax.nn.sigmoid(logit)


def _edge_head(gs, gd, ep, kcol, w, b):
    e = gs.shape[0]
    be = _block(e, 1280)
    dout = w.shape[1]
    return pl.pallas_call(
        _edge_head_body,
        grid=(e // be,),
        in_specs=[_rows(be, H), _rows(be, H),
                  pl.BlockSpec((be, H), lambda i, k=kcol: (i, k)),
                  _full(w.shape), _full(b.shape)],
        out_specs=(_rows(be, dout), _rows(be, dout)),
        out_shape=(jax.ShapeDtypeStruct((e, dout), F32),
                   jax.ShapeDtypeStruct((e, dout), F32)),
    )(gs, gd, ep, w, b)


def _edge_z_body(ex, gq, ep, out):
    out[...] = ex[...] * jnp.maximum(gq[...] + ep[...], 0.0)


def _edge_z(ex, gq, ep, kcol, gcol=0):
    e = gq.shape[0]
    be = _block(e, 1280)
    return pl.pallas_call(
        _edge_z_body,
        grid=(e // be,),
        in_specs=[_rows(be, 1), pl.BlockSpec((be, H), lambda i, g=gcol: (i, g)),
                  pl.BlockSpec((be, H), lambda i, k=kcol: (i, k))],
        out_specs=_rows(be, H),
        out_shape=jax.ShapeDtypeStruct((e, H), F32),
    )(ex, gq, ep)


def _conv_post_body(s, den, xres, w2, b2, g, b, out):
    dv = den[...] + 1e-16
    agg = _dot(s[...] / dv, w2[...]) + (den[...] / dv) * b2[...]
    y = agg + xres[...]
    m = jnp.mean(y, axis=-1, keepdims=True)
    v = jnp.mean((y - m) * (y - m), axis=-1, keepdims=True)
    out[...] = jnp.maximum((y - m) / jnp.sqrt(v + 1e-5) * g[...] + b[...], 0.0)


def _conv_post(s, den, xres, w2, b2, g, b):
    n = s.shape[0]
    bn = _block(n, 2000)
    return pl.pallas_call(
        _conv_post_body,
        grid=(n // bn,),
        in_specs=[_rows(bn, H), _rows(bn, 1), _rows(bn, H), _full(w2.shape),
                  _full(b2.shape), _full(g.shape), _full(b.shape)],
        out_specs=_rows(bn, H),
        out_shape=jax.ShapeDtypeStruct((n, H), F32),
    )(s, den, xres, w2, b2, g, b)


def _node_head_body(x, w1, b1, w2, b2, lo, po):
    h = jnp.maximum(_dot(x[...], w1[...]) + b1[...], 0.0)
    logit = _dot(h, w2[...]) + b2[...]
    lo[...] = logit
    po[...] = jax.nn.softmax(logit, axis=-1)


def _node_head(x, w1, b1, w2, b2):
    n = x.shape[0]
    bn = _block(n, 2000)
    dout = w2.shape[1]
    return pl.pallas_call(
        _node_head_body,
        grid=(n // bn,),
        in_specs=[_rows(bn, H), _full(w1.shape), _full(b1.shape),
                  _full(w2.shape), _full(b2.shape)],
        out_specs=(_rows(bn, dout), _rows(bn, dout)),
        out_shape=(jax.ShapeDtypeStruct((n, dout), F32),
                   jax.ShapeDtypeStruct((n, dout), F32)),
    )(x, w1, b1, w2, b2)


def _sc_gather(table, idx):
    e = idx.shape[0]
    d = table.shape[1]
    per_w = e // _NW
    ch = 128
    while per_w % ch:
        ch //= 2
    n_ch = per_w // ch

    mesh = plsc.VectorSubcoreMesh(core_axis_name="c", subcore_axis_name="s")

    @functools.partial(
        pl.kernel,
        out_type=jax.ShapeDtypeStruct((e, d), F32),
        mesh=mesh,
        compiler_params=pltpu.CompilerParams(needs_layout_passes=False),
        scratch_types=[
            pltpu.VMEM((2, ch), jnp.int32),
            pltpu.VMEM((2, ch, d), F32),
            pltpu.SemaphoreType.DMA((2,)),
        ],
    )
    def k(table_hbm, idx_hbm, out_hbm, idx_v, rows_v, sems):
        wid = lax.axis_index("s") * _NC + lax.axis_index("c")
        base = wid * per_w

        def start(j, slot):
            pltpu.sync_copy(idx_hbm.at[pl.ds(base + j * ch, ch)],
                            idx_v.at[slot])
            pltpu.async_copy(table_hbm.at[idx_v.at[slot]], rows_v.at[slot],
                             sems.at[slot])

        start(0, 0)

        def body(j, _):
            slot = lax.rem(j, 2)
            nxt = 1 - slot

            @pl.when(j + 1 < n_ch)
            def _():
                start(j + 1, nxt)

            pltpu.make_async_copy(table_hbm.at[idx_v.at[slot]],
                                  rows_v.at[slot], sems.at[slot]).wait()
            pltpu.sync_copy(rows_v.at[slot],
                            out_hbm.at[pl.ds(base + j * ch, ch)])
            return 0

        lax.fori_loop(0, n_ch, body, 0)

    return k(table, idx)


def _sc_segred(h, dst, n_pad, init, is_max):
    e = h.shape[0]
    rows = n_pad // _NW
    ch = 2000
    n_ch = e // ch
    mesh = plsc.VectorSubcoreMesh(core_axis_name="c", subcore_axis_name="s")

    @functools.partial(
        pl.kernel,
        out_type=jax.ShapeDtypeStruct((n_pad,), F32),
        mesh=mesh,
        compiler_params=pltpu.CompilerParams(needs_layout_passes=False),
        scratch_types=[
            pltpu.VMEM((rows * 16,), F32),
            pltpu.VMEM((ch,), F32),
            pltpu.VMEM((ch,), jnp.int32),
            pltpu.VMEM((rows,), F32),
        ],
    )
    def k(h_hbm, dst_hbm, m_hbm, tab, h_v, d_v, m_v):
        wid = lax.axis_index("s") * _NC + lax.axis_index("c")
        base = wid * rows
        lane = lax.iota(jnp.int32, 16)
        neg = jnp.full((16,), init, F32)
        op = jnp.maximum if is_max else jnp.add

        def initb(i, _):
            tab[pl.ds(i * 16, 16)] = neg
            return 0
        lax.fori_loop(0, rows, initb, 0)

        def chunk(jc, _):
            pltpu.sync_copy(h_hbm.at[pl.ds(jc * ch, ch)], h_v)
            pltpu.sync_copy(dst_hbm.at[pl.ds(jc * ch, ch)], d_v)

            def step(t, _):
                dv = d_v[pl.ds(t * 16, 16)]
                hv = h_v[pl.ds(t * 16, 16)]
                rel = dv - base
                inr = (rel >= 0) & (rel < rows)
                lidx = jnp.where(inr, rel * 16 + lane, lane)
                cur = plsc.load_gather(tab, [lidx])
                plsc.store_scatter(tab, [lidx], op(cur, hv), mask=inr)
                return 0
            lax.fori_loop(0, ch // 16, step, 0)
            return 0
        lax.fori_loop(0, n_ch, chunk, 0)

        def red(g, _):
            nid = (g * 16 + lane) * 16
            acc = neg
            for kk in range(16):
                acc = op(acc, plsc.load_gather(tab, [nid + kk]))
            m_v[pl.ds(g * 16, 16)] = acc
            return 0
        lax.fori_loop(0, rows // 16, red, 0)
        pltpu.sync_copy(m_v, m_hbm.at[pl.ds(base, rows)])

    return k(h, dst)


def _sc_exp(h, dst, m):
    e = h.shape[0]
    n_pad = m.shape[0]
    per_w = e // _NW
    ch = 2000
    n_ch = per_w // ch
    mesh = plsc.VectorSubcoreMesh(core_axis_name="c", subcore_axis_name="s")

    @functools.partial(
        pl.kernel,
        out_type=jax.ShapeDtypeStruct((e,), F32),
        mesh=mesh,
        compiler_params=pltpu.CompilerParams(needs_layout_passes=False),
        scratch_types=[
            pltpu.VMEM((n_pad,), F32),
            pltpu.VMEM((ch,), F32),
            pltpu.VMEM((ch,), jnp.int32),
            pltpu.VMEM((ch,), F32),
        ],
    )
    def k(h_hbm, dst_hbm, m_hbm, ex_hbm, m_v, h_v, d_v, ex_v):
        wid = lax.axis_index("s") * _NC + lax.axis_index("c")
        base = wid * per_w
        pltpu.sync_copy(m_hbm, m_v)

        def chunk(jc, _):
            e0 = base + jc * ch
            pltpu.sync_copy(h_hbm.at[pl.ds(e0, ch)], h_v)
            pltpu.sync_copy(dst_hbm.at[pl.ds(e0, ch)], d_v)

            def step(t, _):
                dv = d_v[pl.ds(t * 16, 16)]
                hv = h_v[pl.ds(t * 16, 16)]
                mg = plsc.load_gather(m_v, [dv])
                ex_v[pl.ds(t * 16, 16)] = jnp.exp(hv - mg)
                return 0
            lax.fori_loop(0, ch // 16, step, 0)
            pltpu.sync_copy(ex_v, ex_hbm.at[pl.ds(e0, ch)])
            return 0
        lax.fori_loop(0, n_ch, chunk, 0)

    return k(h, dst, m)


def _sc_scatter_rows(z, dst, n_pad):
    e, d = z.shape
    half = n_pad // 2
    acc_rows = half + 128
    stripe = acc_rows // _NS
    ch = 80
    n_ch = e // _NS // ch
    mesh = plsc.VectorSubcoreMesh(core_axis_name="c", subcore_axis_name="s")

    @functools.partial(
        pl.kernel,
        out_type=jax.ShapeDtypeStruct((n_pad, d), F32),
        mesh=mesh,
        compiler_params=pltpu.CompilerParams(needs_layout_passes=False),
        scratch_types=[
            pltpu.VMEM_SHARED((acc_rows, d), F32),
            pltpu.VMEM((ch, d), F32),
            pltpu.VMEM((ch,), jnp.int32),
        ],
    )
    def k(z_hbm, dst_hbm, zeros_hbm, out_hbm, acc, z_v, idx_v):
        sid = lax.axis_index("s")
        c = lax.axis_index("c")
        base_n = c * half
        lane = lax.iota(jnp.int32, 16)

        pltpu.sync_copy(zeros_hbm, acc.at[pl.ds(sid * stripe, stripe)])
        plsc.subcore_barrier()

        def chunk(jc, _):
            e0 = (sid * n_ch + jc) * ch
            pltpu.sync_copy(dst_hbm.at[pl.ds(e0, ch)], idx_v)
            pltpu.sync_copy(z_hbm.at[pl.ds(e0, ch)], z_v)

            def fix(t, _):
                v = idx_v[pl.ds(t * 16, 16)]
                rel = v - base_n
                inr = (rel >= 0) & (rel < half)
                idx_v[pl.ds(t * 16, 16)] = jnp.where(inr, rel, half + lane)
                return 0
            lax.fori_loop(0, ch // 16, fix, 0)
            pltpu.sync_copy(z_v, acc.at[idx_v], add=True)
            return 0
        lax.fori_loop(0, n_ch, chunk, 0)
        plsc.subcore_barrier()

        def wb(i, _):
            r0 = sid * (half // _NS) + i * ch
            pltpu.sync_copy(acc.at[pl.ds(r0, ch)], z_v)
            pltpu.sync_copy(z_v, out_hbm.at[pl.ds(base_n + r0, ch)])
            return 0
        lax.fori_loop(0, half // _NS // ch, wb, 0)

    return k(z, dst, jnp.zeros((stripe, d), F32))


def _conv_stage(xn, EP, k_att, k_msg, src, dst, p, pre, ln_g, ln_b):
    n = xn.shape[0]
    proj_d = _lin(xn, p[pre + '_aW1'][:H])
    wcat = jnp.concatenate([p[pre + '_aW1'][H:2 * H],
                            p[pre + '_mW1'][:H]], axis=1)
    proj_s = _lin(xn, wcat)
    gd = _sc_gather(proj_d, dst)
    gs = _sc_gather(proj_s, src)
    aw2 = jnp.pad(p[pre + '_aW2'], ((0, 0), (0, 7)))
    ab2 = jnp.pad(p[pre + '_ab2'], (0, 7))[None]
    h = _edge_h(gd, gs, EP, k_att, aw2, ab2, gcol=0)[:, 0]
    n_pad = -(-n // (_NW * 80)) * (_NW * 80)
    m = _sc_segred(h, dst, n_pad, -1e30, True)
    ex = _sc_exp(h, dst, m)
    z = _edge_z(ex[:, None], gs, EP, k_msg, gcol=1)
    s = jax.ops.segment_sum(z, dst, num_segments=n)
    den = _sc_segred(ex, dst, n_pad, 0.0, False)[:n, None]
    return _conv_post(s, den, xn, p[pre + '_mW2'], p[pre + '_mb2'][None],
                      ln_g[None], ln_b[None])


def kernel(x_in, edge_index, edge_attr, params):
    p = params
    src = edge_index[0]
    dst = edge_index[1]

    wu = jnp.concatenate([p['c1_aW1'][2 * H:], p['c1_mW1'][H:],
                          p['c2_aW1'][2 * H:], p['c2_mW1'][H:],
                          p['eh_W1'][2 * H:]], axis=1)
    bu = jnp.concatenate([p['c1_ab1'], p['c1_mb1'], p['c2_ab1'],
                          p['c2_mb1'], p['eh_b1']])[None]
    wf, bf = _weight_fuse(p['ee_W2'], p['ee_b2'][None], wu, bu)
    EP = _edge_proj(edge_attr, p['ee_W1'], p['ee_b1'][None], wf, bf)

    x = _node_enc(x_in, p['dummy'][None], p['ne_W1'], p['ne_b1'][None],
                  p['ne_W2'], p['ne_b2'][None])

    x1 = _conv_stage(x, EP, 0, 1, src, dst, p, 'c1', p['ln1_g'], p['ln1_b'])
    x2 = _conv_stage(x1, EP, 2, 3, src, dst, p, 'c2', p['ln2_g'], p['ln2_b'])

    nw2 = jnp.pad(p['nh_W2'], ((0, 0), (0, 1)))
    nb2 = jnp.concatenate([p['nh_b2'], jnp.full((1,), -1e30, F32)])[None]
    node_logits8, node_probs8 = _node_head(x2, p['nh_W1'], p['nh_b1'][None],
                                           nw2, nb2)
    node_logits = node_logits8[:, :7]
    node_probs = node_probs8[:, :7]

    proj_a = _lin(x2, p['eh_W1'][:H])
    proj_b = _lin(x2, p['eh_W1'][H:2 * H])
    ga = _sc_gather(proj_a, src)
    gb = _sc_gather(proj_b, dst)
    ew2 = jnp.pad(p['eh_W2'], ((0, 0), (0, 7)))
    eb2 = jnp.pad(p['eh_b2'], (0, 7))[None]
    el8, ep8 = _edge_head(ga, gb, EP, 4, ew2, eb2)
    edge_logits = el8[:, :1]
    edge_probs = ep8[:, :1]

    return (node_logits, edge_logits, node_probs, edge_probs)
